# baseline (speedup 1.0000x reference)
import functools

import jax
import jax.numpy as jnp
from jax import lax
from jax.experimental import pallas as pl
from jax.experimental.pallas import tpu as pltpu
from jax.experimental.pallas import tpu_sc as plsc

F32 = jnp.float32

N = 10000
E = 320000
CHUNK = 128
NSC = 2
NTILE = 16
NW = NSC * NTILE
NGROUPS = E // (8 * CHUNK)
REMBASE = NGROUPS * 8 * CHUNK
REM_CHUNKS = (E - REMBASE) // CHUNK
GROUPS_LO = NGROUPS // NW
EXTRA_TILES = NGROUPS - NW * GROUPS_LO
ROWS_PER_TILE = 624
ROWS_REM = N - NTILE * ROWS_PER_TILE


def _sc_mesh():
    return plsc.VectorSubcoreMesh(core_axis_name="c", subcore_axis_name="s")


def _sc_msg(dstg, dstr, me12, zA):
    @functools.partial(
        pl.kernel,
        out_type=jax.ShapeDtypeStruct((NSC, N, 32), F32),
        mesh=_sc_mesh(),
        compiler_params=pltpu.CompilerParams(use_tc_tiling_on_sc=False),
        scratch_types=[
            pltpu.VMEM((8, CHUNK), jnp.int32),
            pltpu.VMEM((8 * CHUNK, 32), F32),
            pltpu.VMEM((CHUNK,), jnp.int32),
            pltpu.VMEM((CHUNK, 32), F32),
            pltpu.VMEM_SHARED((N, 32), F32),
            pltpu.SemaphoreType.DMA,
        ],
    )
    def k(dst_h, dstr_h, me_h, zA_h, outA, idx_d, mbuf, idx_dr, mrem, accA, sem):
        c = lax.axis_index("c")
        s = lax.axis_index("s")
        w = c * NTILE + s

        r0 = s * ROWS_PER_TILE
        pltpu.sync_copy(zA_h.at[pl.ds(r0, ROWS_PER_TILE)],
                        accA.at[pl.ds(r0, ROWS_PER_TILE)])

        @pl.when(s == 0)
        def _():
            pltpu.sync_copy(zA_h.at[pl.ds(NTILE * ROWS_PER_TILE, ROWS_REM)],
                            accA.at[pl.ds(NTILE * ROWS_PER_TILE, ROWS_REM)])

        plsc.subcore_barrier()

        def do_group(g):
            pltpu.sync_copy(dst_h.at[g], idx_d)
            pltpu.sync_copy(me_h.at[pl.ds(g * (8 * CHUNK), 8 * CHUNK)], mbuf)
            for j in range(8):
                pltpu.sync_copy(mbuf.at[pl.ds(j * CHUNK, CHUNK)],
                                accA.at[idx_d.at[j]], add=True)

        gstart = w * GROUPS_LO + jnp.minimum(w, EXTRA_TILES)

        def body(g, carry):
            do_group(gstart + g)
            return carry

        lax.fori_loop(0, GROUPS_LO, body, 0)

        @pl.when(w < EXTRA_TILES)
        def _():
            do_group(gstart + GROUPS_LO)

        @pl.when(jnp.logical_and(w >= EXTRA_TILES, w < EXTRA_TILES + REM_CHUNKS))
        def _():
            r = w - EXTRA_TILES
            pltpu.sync_copy(dstr_h.at[r, 0], idx_dr)
            pltpu.sync_copy(me_h.at[pl.ds(REMBASE + r * CHUNK, CHUNK)], mrem)
            pltpu.sync_copy(mrem, accA.at[idx_dr], add=True)

        plsc.subcore_barrier()
        pltpu.sync_copy(accA.at[pl.ds(r0, ROWS_PER_TILE)],
                        outA.at[c, pl.ds(r0, ROWS_PER_TILE)])

        @pl.when(s == 0)
        def _():
            pltpu.sync_copy(accA.at[pl.ds(NTILE * ROWS_PER_TILE, ROWS_REM)],
                            outA.at[c, pl.ds(NTILE * ROWS_PER_TILE, ROWS_REM)])

    return k(dstg, dstr, me12, zA)


def _sc_pass2(src3d, dstg, srcr, dstr, xt2, zB):
    @functools.partial(
        pl.kernel,
        out_type=jax.ShapeDtypeStruct((NSC, N, 16), F32),
        mesh=_sc_mesh(),
        compiler_params=pltpu.CompilerParams(use_tc_tiling_on_sc=False),
        scratch_types=[
            pltpu.VMEM((8, CHUNK), jnp.int32),
            pltpu.VMEM((8, CHUNK), jnp.int32),
            pltpu.VMEM((8 * CHUNK, 16), F32),
            pltpu.VMEM((CHUNK,), jnp.int32),
            pltpu.VMEM((CHUNK,), jnp.int32),
            pltpu.VMEM((CHUNK, 16), F32),
            pltpu.VMEM_SHARED((N, 16), F32),
            pltpu.SemaphoreType.DMA,
        ],
    )
    def k(src_h, dst_h, srcr_h, dstr_h, xt_h, zB_h, outB,
          idx_s, idx_d, gbuf, idx_sr, idx_dr, grem, accB, sem):
        c = lax.axis_index("c")
        s = lax.axis_index("s")
        w = c * NTILE + s

        r0 = s * ROWS_PER_TILE
        pltpu.sync_copy(zB_h.at[pl.ds(r0, ROWS_PER_TILE)],
                        accB.at[pl.ds(r0, ROWS_PER_TILE)])

        @pl.when(s == 0)
        def _():
            pltpu.sync_copy(zB_h.at[pl.ds(NTILE * ROWS_PER_TILE, ROWS_REM)],
                            accB.at[pl.ds(NTILE * ROWS_PER_TILE, ROWS_REM)])

        plsc.subcore_barrier()

        def do_group(g):
            pltpu.sync_copy(src_h.at[g], idx_s)
            pltpu.sync_copy(dst_h.at[g], idx_d)
            cps = [pltpu.async_copy(xt_h.at[idx_s.at[j]],
                                    gbuf.at[pl.ds(j * CHUNK, CHUNK)], sem)
                   for j in range(8)]
            for cp in cps:
                cp.wait()
            for j in range(8):
                pltpu.sync_copy(gbuf.at[pl.ds(j * CHUNK, CHUNK)],
                                accB.at[idx_d.at[j]], add=True)

        gstart = w * GROUPS_LO + jnp.minimum(w, EXTRA_TILES)

        def body(g, carry):
            do_group(gstart + g)
            return carry

        lax.fori_loop(0, GROUPS_LO, body, 0)

        @pl.when(w < EXTRA_TILES)
        def _():
            do_group(gstart + GROUPS_LO)

        @pl.when(jnp.logical_and(w >= EXTRA_TILES, w < EXTRA_TILES + REM_CHUNKS))
        def _():
            r = w - EXTRA_TILES
            pltpu.sync_copy(srcr_h.at[r, 0], idx_sr)
            pltpu.sync_copy(dstr_h.at[r, 0], idx_dr)
            pltpu.async_copy(xt_h.at[idx_sr], grem, sem).wait()
            pltpu.sync_copy(grem, accB.at[idx_dr], add=True)

        plsc.subcore_barrier()
        pltpu.sync_copy(accB.at[pl.ds(r0, ROWS_PER_TILE)],
                        outB.at[c, pl.ds(r0, ROWS_PER_TILE)])

        @pl.when(s == 0)
        def _():
            pltpu.sync_copy(accB.at[pl.ds(NTILE * ROWS_PER_TILE, ROWS_REM)],
                            outB.at[c, pl.ds(NTILE * ROWS_PER_TILE, ROWS_REM)])

    return k(src3d, dstg, srcr, dstr, xt2, zB)


def _edge_mlp_body(ea_ref, A_ref, a_ref, B_ref, cbias_ref,
                   x_ref, ntw_ref, ntb_ref, out_ref, xt1_ref):
    h = jnp.maximum(ea_ref[...] @ A_ref[...] + a_ref[...], 0.0)
    out_ref[...] = h @ B_ref[...] + cbias_ref[...]
    xt1_ref[...] = x_ref[...] @ ntw_ref[...] + ntb_ref[...]


def _tc_edge_mlp(ea_packed, A12k, a12k, B12k, c12k, x, ntw, ntb):
    BE = 4000
    BN = N // 10
    EP = E // 8
    return pl.pallas_call(
        _edge_mlp_body,
        grid=(EP // BE,),
        in_specs=[
            pl.BlockSpec((BE, 128), lambda i: (i, 0)),
            pl.BlockSpec((128, 256), lambda i: (0, 0)),
            pl.BlockSpec((1, 256), lambda i: (0, 0)),
            pl.BlockSpec((256, 256), lambda i: (0, 0)),
            pl.BlockSpec((1, 256), lambda i: (0, 0)),
            pl.BlockSpec((BN, 128), lambda i: (i, 0)),
            pl.BlockSpec((128, 16), lambda i: (0, 0)),
            pl.BlockSpec((1, 16), lambda i: (0, 0)),
        ],
        out_specs=(pl.BlockSpec((BE, 256), lambda i: (i, 0)),
                   pl.BlockSpec((BN, 16), lambda i: (i, 0))),
        out_shape=(jax.ShapeDtypeStruct((EP, 256), F32),
                   jax.ShapeDtypeStruct((N, 16), F32)),
    )(ea_packed, A12k, a12k, B12k, c12k, x, ntw, ntb)


def _xt1_body(x_ref, w_ref, b_ref, out_ref):
    out_ref[...] = x_ref[...] @ w_ref[...] + b_ref[...]


def _tc_xt1(x, w, b):
    BN = 2000
    return pl.pallas_call(
        _xt1_body,
        grid=(N // BN,),
        in_specs=[
            pl.BlockSpec((BN, 128), lambda i: (i, 0)),
            pl.BlockSpec((128, 16), lambda i: (0, 0)),
            pl.BlockSpec((1, 16), lambda i: (0, 0)),
        ],
        out_specs=pl.BlockSpec((BN, 16), lambda i: (i, 0)),
        out_shape=jax.ShapeDtypeStruct((N, 16), F32),
    )(x, w, b)


def _combine1_body(outA_ref, outB_ref, xt1_ref, um1_ref, um1b_ref,
                   um2_ref, um2b_ref, nt2_ref, nt2b_ref, xt2_ref, sm2_ref):
    accA = outA_ref[0] + outA_ref[1]
    aggr1 = outB_ref[0] + outB_ref[1] + accA[:, :16]
    sm2_ref[...] = accA[:, 16:]
    xt1 = xt1_ref[...]
    h = jnp.maximum(xt1 @ um1_ref[:16, :] + aggr1 @ um1_ref[16:, :] + um1b_ref[...], 0.0)
    x1 = jnp.maximum(h @ um2_ref[...] + um2b_ref[...], 0.0)
    xt2_ref[...] = x1 @ nt2_ref[...] + nt2b_ref[...]


def _tc_combine1(outA, outB, xt1, um1, um1b, um2, um2b, nt2, nt2b):
    return pl.pallas_call(
        _combine1_body,
        out_shape=(jax.ShapeDtypeStruct((N, 16), F32),
                   jax.ShapeDtypeStruct((N, 16), F32)),
    )(outA, outB, xt1, um1, um1b, um2, um2b, nt2, nt2b)


def _combine2_body(outB2_ref, sm2_ref, xt2_ref, um1_ref, um1b_ref,
                   um2_ref, um2b_ref, fcw_ref, fcb_ref, out_ref):
    aggr2 = outB2_ref[0] + outB2_ref[1] + sm2_ref[...]
    xt2 = xt2_ref[...]
    h = jnp.maximum(xt2 @ um1_ref[:16, :] + aggr2 @ um1_ref[16:, :] + um1b_ref[...], 0.0)
    x2 = jnp.maximum(h @ um2_ref[...] + um2b_ref[...], 0.0)
    pooled = jnp.sum(x2, axis=0, keepdims=True) * (1.0 / N)
    out_ref[...] = pooled @ fcw_ref[...] + fcb_ref[...]


def _tc_combine2(outB2, sm2, xt2, um1, um1b, um2, um2b, fcw, fcb):
    return pl.pallas_call(
        _combine2_body,
        out_shape=jax.ShapeDtypeStruct((1, 1), F32),
    )(outB2, sm2, xt2, um1, um1b, um2, um2b, fcw, fcb)


def kernel(x, edge_index, edge_attr, batch, params):
    p1, p2 = params['conv1'], params['conv2']

    A1 = p1['et_w'] @ p1['em1_w']
    a1 = p1['et_b'] @ p1['em1_w'] + p1['em1_b']
    A2 = p2['et_w'] @ p2['em1_w']
    a2 = p2['et_b'] @ p2['em1_w'] + p2['em1_b']
    A12 = jnp.concatenate([A1, A2], axis=1)
    a12 = jnp.concatenate([a1, a2])[None, :]
    Z16 = jnp.zeros((16, 16), F32)
    B12 = jnp.block([[p1['em2_w'], Z16], [Z16, p2['em2_w']]])
    c12 = jnp.concatenate([p1['em2_b'], p2['em2_b']])[None, :]
    I8 = jnp.eye(8, dtype=F32)
    A12k = jnp.kron(I8, A12)
    B12k = jnp.kron(I8, B12)
    a12k = jnp.tile(a12, (1, 8))
    c12k = jnp.tile(c12, (1, 8))

    src, dst = edge_index[0], edge_index[1]
    src3d = src[:REMBASE].reshape(NGROUPS, 8, CHUNK)
    dst3d = dst[:REMBASE].reshape(NGROUPS, 8, CHUNK)
    srcr = src[REMBASE:].reshape(REM_CHUNKS, 1, CHUNK)
    dstr = dst[REMBASE:].reshape(REM_CHUNKS, 1, CHUNK)
    zA = jnp.zeros((N, 32), F32)
    zB = jnp.zeros((N, 16), F32)

    xt1 = _tc_xt1(x, p1['nt_w'], p1['nt_b'][None, :])
    outB = _sc_pass2(src3d, dst3d, srcr, dstr, xt1, zB)

    me12p, _ = _tc_edge_mlp(edge_attr.reshape(E // 8, 128),
                            A12k, a12k, B12k, c12k,
                            x, p1['nt_w'], p1['nt_b'][None, :])
    me12 = me12p.reshape(E, 32)
    outA = _sc_msg(dst3d, dstr, me12, zA)

    xt2, sm2 = _tc_combine1(outA, outB, xt1,
                            p1['um1_w'], p1['um1_b'][None, :],
                            p1['um2_w'], p1['um2_b'][None, :],
                            p2['nt_w'], p2['nt_b'][None, :])

    outB2 = _sc_pass2(src3d, dst3d, srcr, dstr, xt2, zB)

    out = _tc_combine2(outB2, sm2, xt2,
                       p2['um1_w'], p2['um1_b'][None, :],
                       p2['um2_w'], p2['um2_b'][None, :],
                       params['fc_w'], params['fc_b'][None, :])
    return out


# speedup vs baseline: 1.0337x; 1.0337x over previous
import functools

import jax
import jax.numpy as jnp
from jax import lax
from jax.experimental import pallas as pl
from jax.experimental.pallas import tpu as pltpu
from jax.experimental.pallas import tpu_sc as plsc

F32 = jnp.float32

N = 10000
E = 320000
CHUNK = 128
NSC = 2
NTILE = 16
NW = NSC * NTILE
NGROUPS = E // (8 * CHUNK)
REMBASE = NGROUPS * 8 * CHUNK
REM_CHUNKS = (E - REMBASE) // CHUNK
GROUPS_LO = NGROUPS // NW
EXTRA_TILES = NGROUPS - NW * GROUPS_LO
ROWS_PER_TILE = 624
ROWS_REM = N - NTILE * ROWS_PER_TILE


def _sc_mesh():
    return plsc.VectorSubcoreMesh(core_axis_name="c", subcore_axis_name="s")


def _sc_pass1(src3d, dstg, srcr, dstr, xtp, me12, zA):
    @functools.partial(
        pl.kernel,
        out_type=jax.ShapeDtypeStruct((NSC, N, 32), F32),
        mesh=_sc_mesh(),
        compiler_params=pltpu.CompilerParams(use_tc_tiling_on_sc=False),
        scratch_types=[
            pltpu.VMEM((8, CHUNK), jnp.int32),
            pltpu.VMEM((8, CHUNK), jnp.int32),
            pltpu.VMEM((8 * CHUNK, 32), F32),
            pltpu.VMEM((8 * CHUNK, 32), F32),
            pltpu.VMEM((CHUNK,), jnp.int32),
            pltpu.VMEM((CHUNK,), jnp.int32),
            pltpu.VMEM((CHUNK, 32), F32),
            pltpu.VMEM((CHUNK, 32), F32),
            pltpu.VMEM_SHARED((N, 32), F32),
            pltpu.SemaphoreType.DMA,
        ],
    )
    def k(src_h, dst_h, srcr_h, dstr_h, xt_h, me_h, zA_h, outA,
          idx_s, idx_d, gbuf, mbuf, idx_sr, idx_dr, grem, mrem,
          accA, sem):
        c = lax.axis_index("c")
        s = lax.axis_index("s")
        w = c * NTILE + s

        r0 = s * ROWS_PER_TILE
        pltpu.sync_copy(zA_h.at[pl.ds(r0, ROWS_PER_TILE)],
                        accA.at[pl.ds(r0, ROWS_PER_TILE)])

        @pl.when(s == 0)
        def _():
            pltpu.sync_copy(zA_h.at[pl.ds(NTILE * ROWS_PER_TILE, ROWS_REM)],
                            accA.at[pl.ds(NTILE * ROWS_PER_TILE, ROWS_REM)])

        plsc.subcore_barrier()

        def do_group(g):
            pltpu.sync_copy(src_h.at[g], idx_s)
            pltpu.sync_copy(dst_h.at[g], idx_d)
            pltpu.sync_copy(me_h.at[pl.ds(g * (8 * CHUNK), 8 * CHUNK)], mbuf)
            cps = [pltpu.async_copy(xt_h.at[idx_s.at[j]],
                                    gbuf.at[pl.ds(j * CHUNK, CHUNK)], sem)
                   for j in range(8)]
            for cp in cps:
                cp.wait()
            for j in range(8):
                pltpu.sync_copy(gbuf.at[pl.ds(j * CHUNK, CHUNK)],
                                accA.at[idx_d.at[j]], add=True)
                pltpu.sync_copy(mbuf.at[pl.ds(j * CHUNK, CHUNK)],
                                accA.at[idx_d.at[j]], add=True)

        gstart = w * GROUPS_LO + jnp.minimum(w, EXTRA_TILES)

        def body(g, carry):
            do_group(gstart + g)
            return carry

        lax.fori_loop(0, GROUPS_LO, body, 0)

        @pl.when(w < EXTRA_TILES)
        def _():
            do_group(gstart + GROUPS_LO)

        @pl.when(jnp.logical_and(w >= EXTRA_TILES, w < EXTRA_TILES + REM_CHUNKS))
        def _():
            r = w - EXTRA_TILES
            pltpu.sync_copy(srcr_h.at[r, 0], idx_sr)
            pltpu.sync_copy(dstr_h.at[r, 0], idx_dr)
            pltpu.sync_copy(me_h.at[pl.ds(REMBASE + r * CHUNK, CHUNK)], mrem)
            pltpu.async_copy(xt_h.at[idx_sr], grem, sem).wait()
            pltpu.sync_copy(grem, accA.at[idx_dr], add=True)
            pltpu.sync_copy(mrem, accA.at[idx_dr], add=True)

        plsc.subcore_barrier()
        pltpu.sync_copy(accA.at[pl.ds(r0, ROWS_PER_TILE)],
                        outA.at[c, pl.ds(r0, ROWS_PER_TILE)])

        @pl.when(s == 0)
        def _():
            pltpu.sync_copy(accA.at[pl.ds(NTILE * ROWS_PER_TILE, ROWS_REM)],
                            outA.at[c, pl.ds(NTILE * ROWS_PER_TILE, ROWS_REM)])

    return k(src3d, dstg, srcr, dstr, xtp, me12, zA)


def _sc_pass2(src3d, dstg, srcr, dstr, xt2, zB):
    @functools.partial(
        pl.kernel,
        out_type=jax.ShapeDtypeStruct((NSC, N, 16), F32),
        mesh=_sc_mesh(),
        compiler_params=pltpu.CompilerParams(use_tc_tiling_on_sc=False),
        scratch_types=[
            pltpu.VMEM((8, CHUNK), jnp.int32),
            pltpu.VMEM((8, CHUNK), jnp.int32),
            pltpu.VMEM((8 * CHUNK, 16), F32),
            pltpu.VMEM((CHUNK,), jnp.int32),
            pltpu.VMEM((CHUNK,), jnp.int32),
            pltpu.VMEM((CHUNK, 16), F32),
            pltpu.VMEM_SHARED((N, 16), F32),
            pltpu.SemaphoreType.DMA,
        ],
    )
    def k(src_h, dst_h, srcr_h, dstr_h, xt_h, zB_h, outB,
          idx_s, idx_d, gbuf, idx_sr, idx_dr, grem, accB, sem):
        c = lax.axis_index("c")
        s = lax.axis_index("s")
        w = c * NTILE + s

        r0 = s * ROWS_PER_TILE
        pltpu.sync_copy(zB_h.at[pl.ds(r0, ROWS_PER_TILE)],
                        accB.at[pl.ds(r0, ROWS_PER_TILE)])

        @pl.when(s == 0)
        def _():
            pltpu.sync_copy(zB_h.at[pl.ds(NTILE * ROWS_PER_TILE, ROWS_REM)],
                            accB.at[pl.ds(NTILE * ROWS_PER_TILE, ROWS_REM)])

        plsc.subcore_barrier()

        def do_group(g):
            pltpu.sync_copy(src_h.at[g], idx_s)
            pltpu.sync_copy(dst_h.at[g], idx_d)
            cps = [pltpu.async_copy(xt_h.at[idx_s.at[j]],
                                    gbuf.at[pl.ds(j * CHUNK, CHUNK)], sem)
                   for j in range(8)]
            for cp in cps:
                cp.wait()
            for j in range(8):
                pltpu.sync_copy(gbuf.at[pl.ds(j * CHUNK, CHUNK)],
                                accB.at[idx_d.at[j]], add=True)

        gstart = w * GROUPS_LO + jnp.minimum(w, EXTRA_TILES)

        def body(g, carry):
            do_group(gstart + g)
            return carry

        lax.fori_loop(0, GROUPS_LO, body, 0)

        @pl.when(w < EXTRA_TILES)
        def _():
            do_group(gstart + GROUPS_LO)

        @pl.when(jnp.logical_and(w >= EXTRA_TILES, w < EXTRA_TILES + REM_CHUNKS))
        def _():
            r = w - EXTRA_TILES
            pltpu.sync_copy(srcr_h.at[r, 0], idx_sr)
            pltpu.sync_copy(dstr_h.at[r, 0], idx_dr)
            pltpu.async_copy(xt_h.at[idx_sr], grem, sem).wait()
            pltpu.sync_copy(grem, accB.at[idx_dr], add=True)

        plsc.subcore_barrier()
        pltpu.sync_copy(accB.at[pl.ds(r0, ROWS_PER_TILE)],
                        outB.at[c, pl.ds(r0, ROWS_PER_TILE)])

        @pl.when(s == 0)
        def _():
            pltpu.sync_copy(accB.at[pl.ds(NTILE * ROWS_PER_TILE, ROWS_REM)],
                            outB.at[c, pl.ds(NTILE * ROWS_PER_TILE, ROWS_REM)])

    return k(src3d, dstg, srcr, dstr, xt2, zB)


def _edge_mlp_body(ea_ref, A_ref, a_ref, B_ref, cbias_ref,
                   x_ref, ntw_ref, ntb_ref, out_ref, xt1_ref, xtp_ref):
    h = jnp.maximum(ea_ref[...] @ A_ref[...] + a_ref[...], 0.0)
    out_ref[...] = h @ B_ref[...] + cbias_ref[...]
    xt1 = x_ref[...] @ ntw_ref[...] + ntb_ref[...]
    xt1_ref[...] = xt1
    xtp_ref[...] = jnp.concatenate([xt1, jnp.zeros_like(xt1)], axis=1)


def _tc_edge_mlp(ea_packed, A12k, a12k, B12k, c12k, x, ntw, ntb):
    BE = 4000
    BN = N // 10
    EP = E // 8
    return pl.pallas_call(
        _edge_mlp_body,
        grid=(EP // BE,),
        in_specs=[
            pl.BlockSpec((BE, 128), lambda i: (i, 0)),
            pl.BlockSpec((128, 256), lambda i: (0, 0)),
            pl.BlockSpec((1, 256), lambda i: (0, 0)),
            pl.BlockSpec((256, 256), lambda i: (0, 0)),
            pl.BlockSpec((1, 256), lambda i: (0, 0)),
            pl.BlockSpec((BN, 128), lambda i: (i, 0)),
            pl.BlockSpec((128, 16), lambda i: (0, 0)),
            pl.BlockSpec((1, 16), lambda i: (0, 0)),
        ],
        out_specs=(pl.BlockSpec((BE, 256), lambda i: (i, 0)),
                   pl.BlockSpec((BN, 16), lambda i: (i, 0)),
                   pl.BlockSpec((BN, 32), lambda i: (i, 0))),
        out_shape=(jax.ShapeDtypeStruct((EP, 256), F32),
                   jax.ShapeDtypeStruct((N, 16), F32),
                   jax.ShapeDtypeStruct((N, 32), F32)),
    )(ea_packed, A12k, a12k, B12k, c12k, x, ntw, ntb)


def _xt1_body(x_ref, w_ref, b_ref, out_ref):
    out_ref[...] = x_ref[...] @ w_ref[...] + b_ref[...]


def _tc_xt1(x, w, b):
    BN = 2000
    return pl.pallas_call(
        _xt1_body,
        grid=(N // BN,),
        in_specs=[
            pl.BlockSpec((BN, 128), lambda i: (i, 0)),
            pl.BlockSpec((128, 16), lambda i: (0, 0)),
            pl.BlockSpec((1, 16), lambda i: (0, 0)),
        ],
        out_specs=pl.BlockSpec((BN, 16), lambda i: (i, 0)),
        out_shape=jax.ShapeDtypeStruct((N, 16), F32),
    )(x, w, b)


def _combine1_body(outA_ref, xt1_ref, um1_ref, um1b_ref,
                   um2_ref, um2b_ref, nt2_ref, nt2b_ref, xt2_ref, sm2_ref):
    accA = outA_ref[0] + outA_ref[1]
    aggr1 = accA[:, :16]
    sm2_ref[...] = accA[:, 16:]
    xt1 = xt1_ref[...]
    h = jnp.maximum(xt1 @ um1_ref[:16, :] + aggr1 @ um1_ref[16:, :] + um1b_ref[...], 0.0)
    x1 = jnp.maximum(h @ um2_ref[...] + um2b_ref[...], 0.0)
    xt2_ref[...] = x1 @ nt2_ref[...] + nt2b_ref[...]


def _tc_combine1(outA, xt1, um1, um1b, um2, um2b, nt2, nt2b):
    return pl.pallas_call(
        _combine1_body,
        out_shape=(jax.ShapeDtypeStruct((N, 16), F32),
                   jax.ShapeDtypeStruct((N, 16), F32)),
    )(outA, xt1, um1, um1b, um2, um2b, nt2, nt2b)


def _combine2_body(outB2_ref, sm2_ref, xt2_ref, um1_ref, um1b_ref,
                   um2_ref, um2b_ref, fcw_ref, fcb_ref, out_ref):
    aggr2 = outB2_ref[0] + outB2_ref[1] + sm2_ref[...]
    xt2 = xt2_ref[...]
    h = jnp.maximum(xt2 @ um1_ref[:16, :] + aggr2 @ um1_ref[16:, :] + um1b_ref[...], 0.0)
    x2 = jnp.maximum(h @ um2_ref[...] + um2b_ref[...], 0.0)
    pooled = jnp.sum(x2, axis=0, keepdims=True) * (1.0 / N)
    out_ref[...] = pooled @ fcw_ref[...] + fcb_ref[...]


def _tc_combine2(outB2, sm2, xt2, um1, um1b, um2, um2b, fcw, fcb):
    return pl.pallas_call(
        _combine2_body,
        out_shape=jax.ShapeDtypeStruct((1, 1), F32),
    )(outB2, sm2, xt2, um1, um1b, um2, um2b, fcw, fcb)


def kernel(x, edge_index, edge_attr, batch, params):
    p1, p2 = params['conv1'], params['conv2']

    A1 = p1['et_w'] @ p1['em1_w']
    a1 = p1['et_b'] @ p1['em1_w'] + p1['em1_b']
    A2 = p2['et_w'] @ p2['em1_w']
    a2 = p2['et_b'] @ p2['em1_w'] + p2['em1_b']
    A12 = jnp.concatenate([A1, A2], axis=1)
    a12 = jnp.concatenate([a1, a2])[None, :]
    Z16 = jnp.zeros((16, 16), F32)
    B12 = jnp.block([[p1['em2_w'], Z16], [Z16, p2['em2_w']]])
    c12 = jnp.concatenate([p1['em2_b'], p2['em2_b']])[None, :]
    I8 = jnp.eye(8, dtype=F32)
    A12k = jnp.kron(I8, A12)
    B12k = jnp.kron(I8, B12)
    a12k = jnp.tile(a12, (1, 8))
    c12k = jnp.tile(c12, (1, 8))

    src, dst = edge_index[0], edge_index[1]
    src3d = src[:REMBASE].reshape(NGROUPS, 8, CHUNK)
    dst3d = dst[:REMBASE].reshape(NGROUPS, 8, CHUNK)
    srcr = src[REMBASE:].reshape(REM_CHUNKS, 1, CHUNK)
    dstr = dst[REMBASE:].reshape(REM_CHUNKS, 1, CHUNK)
    zA = jnp.zeros((N, 32), F32)
    zB = jnp.zeros((N, 16), F32)

    me12p, xt1, xtp = _tc_edge_mlp(edge_attr.reshape(E // 8, 128),
                                   A12k, a12k, B12k, c12k,
                                   x, p1['nt_w'], p1['nt_b'][None, :])
    me12 = me12p.reshape(E, 32)

    outA = _sc_pass1(src3d, dst3d, srcr, dstr, xtp, me12, zA)

    xt2, sm2 = _tc_combine1(outA, xt1,
                            p1['um1_w'], p1['um1_b'][None, :],
                            p1['um2_w'], p1['um2_b'][None, :],
                            p2['nt_w'], p2['nt_b'][None, :])

    outB2 = _sc_pass2(src3d, dst3d, srcr, dstr, xt2, zB)

    out = _tc_combine2(outB2, sm2, xt2,
                       p2['um1_w'], p2['um1_b'][None, :],
                       p2['um2_w'], p2['um2_b'][None, :],
                       params['fc_w'], params['fc_b'][None, :])
    return out


# speedup vs baseline: 1.0943x; 1.0586x over previous
import functools

import jax
import jax.numpy as jnp
from jax import lax
from jax.experimental import pallas as pl
from jax.experimental.pallas import tpu as pltpu
from jax.experimental.pallas import tpu_sc as plsc

F32 = jnp.float32

N = 10000
E = 320000
CHUNK = 128
NSC = 2
NTILE = 16
NW = NSC * NTILE
NGROUPS = E // (8 * CHUNK)
REMBASE = NGROUPS * 8 * CHUNK
REM_CHUNKS = (E - REMBASE) // CHUNK
GROUPS_LO = NGROUPS // NW
EXTRA_TILES = NGROUPS - NW * GROUPS_LO
ROWS_PER_TILE = 624
ROWS_REM = N - NTILE * ROWS_PER_TILE


def _sc_mesh():
    return plsc.VectorSubcoreMesh(core_axis_name="c", subcore_axis_name="s")


def _sc_pass1(src3d, dstg, srcr, dstr, xtp, me12, zA):
    @functools.partial(
        pl.kernel,
        out_type=jax.ShapeDtypeStruct((NSC, N, 32), F32),
        mesh=_sc_mesh(),
        compiler_params=pltpu.CompilerParams(use_tc_tiling_on_sc=False),
        scratch_types=[
            pltpu.VMEM((8, CHUNK), jnp.int32),
            pltpu.VMEM((8, CHUNK), jnp.int32),
            pltpu.VMEM((8 * CHUNK, 32), F32),
            pltpu.VMEM((8 * CHUNK, 32), F32),
            pltpu.VMEM((CHUNK,), jnp.int32),
            pltpu.VMEM((CHUNK,), jnp.int32),
            pltpu.VMEM((CHUNK, 32), F32),
            pltpu.VMEM((CHUNK, 32), F32),
            pltpu.VMEM_SHARED((N, 32), F32),
            pltpu.SemaphoreType.DMA,
            pltpu.SemaphoreType.DMA,
        ],
    )
    def k(src_h, dst_h, srcr_h, dstr_h, xt_h, me_h, zA_h, outA,
          idx_s, idx_d, gbuf, mbuf, idx_sr, idx_dr, grem, mrem,
          accA, sem, sem2):
        c = lax.axis_index("c")
        s = lax.axis_index("s")
        w = c * NTILE + s

        r0 = s * ROWS_PER_TILE
        pltpu.sync_copy(zA_h.at[pl.ds(r0, ROWS_PER_TILE)],
                        accA.at[pl.ds(r0, ROWS_PER_TILE)])

        @pl.when(s == 0)
        def _():
            pltpu.sync_copy(zA_h.at[pl.ds(NTILE * ROWS_PER_TILE, ROWS_REM)],
                            accA.at[pl.ds(NTILE * ROWS_PER_TILE, ROWS_REM)])

        plsc.subcore_barrier()

        def do_group(g):
            pltpu.sync_copy(src_h.at[g], idx_s)
            pltpu.sync_copy(dst_h.at[g], idx_d)
            pltpu.sync_copy(me_h.at[pl.ds(g * (8 * CHUNK), 8 * CHUNK)], mbuf)
            cps = [pltpu.async_copy(xt_h.at[idx_s.at[j]],
                                    gbuf.at[pl.ds(j * CHUNK, CHUNK)], sem)
                   for j in range(8)]
            mcps = [pltpu.async_copy(mbuf.at[pl.ds(j * CHUNK, CHUNK)],
                                     accA.at[idx_d.at[j]], sem2, add=True)
                    for j in range(8)]
            for cp in cps:
                cp.wait()
            gcps = [pltpu.async_copy(gbuf.at[pl.ds(j * CHUNK, CHUNK)],
                                     accA.at[idx_d.at[j]], sem2, add=True)
                    for j in range(8)]
            for cp in mcps:
                cp.wait()
            for cp in gcps:
                cp.wait()

        gstart = w * GROUPS_LO + jnp.minimum(w, EXTRA_TILES)

        def body(g, carry):
            do_group(gstart + g)
            return carry

        lax.fori_loop(0, GROUPS_LO, body, 0)

        @pl.when(w < EXTRA_TILES)
        def _():
            do_group(gstart + GROUPS_LO)

        @pl.when(jnp.logical_and(w >= EXTRA_TILES, w < EXTRA_TILES + REM_CHUNKS))
        def _():
            r = w - EXTRA_TILES
            pltpu.sync_copy(srcr_h.at[r, 0], idx_sr)
            pltpu.sync_copy(dstr_h.at[r, 0], idx_dr)
            pltpu.sync_copy(me_h.at[pl.ds(REMBASE + r * CHUNK, CHUNK)], mrem)
            pltpu.async_copy(xt_h.at[idx_sr], grem, sem).wait()
            pltpu.sync_copy(grem, accA.at[idx_dr], add=True)
            pltpu.sync_copy(mrem, accA.at[idx_dr], add=True)

        plsc.subcore_barrier()
        pltpu.sync_copy(accA.at[pl.ds(r0, ROWS_PER_TILE)],
                        outA.at[c, pl.ds(r0, ROWS_PER_TILE)])

        @pl.when(s == 0)
        def _():
            pltpu.sync_copy(accA.at[pl.ds(NTILE * ROWS_PER_TILE, ROWS_REM)],
                            outA.at[c, pl.ds(NTILE * ROWS_PER_TILE, ROWS_REM)])

    return k(src3d, dstg, srcr, dstr, xtp, me12, zA)


def _sc_pass2(src3d, dstg, srcr, dstr, xt2, zB):
    @functools.partial(
        pl.kernel,
        out_type=jax.ShapeDtypeStruct((NSC, N, 16), F32),
        mesh=_sc_mesh(),
        compiler_params=pltpu.CompilerParams(use_tc_tiling_on_sc=False),
        scratch_types=[
            pltpu.VMEM((8, CHUNK), jnp.int32),
            pltpu.VMEM((8, CHUNK), jnp.int32),
            pltpu.VMEM((8 * CHUNK, 16), F32),
            pltpu.VMEM((CHUNK,), jnp.int32),
            pltpu.VMEM((CHUNK,), jnp.int32),
            pltpu.VMEM((CHUNK, 16), F32),
            pltpu.VMEM_SHARED((N, 16), F32),
            pltpu.SemaphoreType.DMA,
            pltpu.SemaphoreType.DMA,
        ],
    )
    def k(src_h, dst_h, srcr_h, dstr_h, xt_h, zB_h, outB,
          idx_s, idx_d, gbuf, idx_sr, idx_dr, grem, accB, sem, sem2):
        c = lax.axis_index("c")
        s = lax.axis_index("s")
        w = c * NTILE + s

        r0 = s * ROWS_PER_TILE
        pltpu.sync_copy(zB_h.at[pl.ds(r0, ROWS_PER_TILE)],
                        accB.at[pl.ds(r0, ROWS_PER_TILE)])

        @pl.when(s == 0)
        def _():
            pltpu.sync_copy(zB_h.at[pl.ds(NTILE * ROWS_PER_TILE, ROWS_REM)],
                            accB.at[pl.ds(NTILE * ROWS_PER_TILE, ROWS_REM)])

        plsc.subcore_barrier()

        def do_group(g):
            pltpu.sync_copy(src_h.at[g], idx_s)
            pltpu.sync_copy(dst_h.at[g], idx_d)
            cps = [pltpu.async_copy(xt_h.at[idx_s.at[j]],
                                    gbuf.at[pl.ds(j * CHUNK, CHUNK)], sem)
                   for j in range(8)]
            for cp in cps:
                cp.wait()
            gcps = [pltpu.async_copy(gbuf.at[pl.ds(j * CHUNK, CHUNK)],
                                     accB.at[idx_d.at[j]], sem2, add=True)
                    for j in range(8)]
            for cp in gcps:
                cp.wait()

        gstart = w * GROUPS_LO + jnp.minimum(w, EXTRA_TILES)

        def body(g, carry):
            do_group(gstart + g)
            return carry

        lax.fori_loop(0, GROUPS_LO, body, 0)

        @pl.when(w < EXTRA_TILES)
        def _():
            do_group(gstart + GROUPS_LO)

        @pl.when(jnp.logical_and(w >= EXTRA_TILES, w < EXTRA_TILES + REM_CHUNKS))
        def _():
            r = w - EXTRA_TILES
            pltpu.sync_copy(srcr_h.at[r, 0], idx_sr)
            pltpu.sync_copy(dstr_h.at[r, 0], idx_dr)
            pltpu.async_copy(xt_h.at[idx_sr], grem, sem).wait()
            pltpu.sync_copy(grem, accB.at[idx_dr], add=True)

        plsc.subcore_barrier()
        pltpu.sync_copy(accB.at[pl.ds(r0, ROWS_PER_TILE)],
                        outB.at[c, pl.ds(r0, ROWS_PER_TILE)])

        @pl.when(s == 0)
        def _():
            pltpu.sync_copy(accB.at[pl.ds(NTILE * ROWS_PER_TILE, ROWS_REM)],
                            outB.at[c, pl.ds(NTILE * ROWS_PER_TILE, ROWS_REM)])

    return k(src3d, dstg, srcr, dstr, xt2, zB)


def _edge_mlp_body(ea_ref, A_ref, a_ref, B_ref, cbias_ref,
                   x_ref, ntw_ref, ntb_ref, out_ref, xt1_ref, xtp_ref):
    h = jnp.maximum(ea_ref[...] @ A_ref[...] + a_ref[...], 0.0)
    out_ref[...] = h @ B_ref[...] + cbias_ref[...]
    xt1 = x_ref[...] @ ntw_ref[...] + ntb_ref[...]
    xt1_ref[...] = xt1
    xtp_ref[...] = jnp.concatenate([xt1, jnp.zeros_like(xt1)], axis=1)


def _tc_edge_mlp(ea_packed, A12k, a12k, B12k, c12k, x, ntw, ntb):
    BE = 4000
    BN = N // 10
    EP = E // 8
    return pl.pallas_call(
        _edge_mlp_body,
        grid=(EP // BE,),
        in_specs=[
            pl.BlockSpec((BE, 128), lambda i: (i, 0)),
            pl.BlockSpec((128, 256), lambda i: (0, 0)),
            pl.BlockSpec((1, 256), lambda i: (0, 0)),
            pl.BlockSpec((256, 256), lambda i: (0, 0)),
            pl.BlockSpec((1, 256), lambda i: (0, 0)),
            pl.BlockSpec((BN, 128), lambda i: (i, 0)),
            pl.BlockSpec((128, 16), lambda i: (0, 0)),
            pl.BlockSpec((1, 16), lambda i: (0, 0)),
        ],
        out_specs=(pl.BlockSpec((BE, 256), lambda i: (i, 0)),
                   pl.BlockSpec((BN, 16), lambda i: (i, 0)),
                   pl.BlockSpec((BN, 32), lambda i: (i, 0))),
        out_shape=(jax.ShapeDtypeStruct((EP, 256), F32),
                   jax.ShapeDtypeStruct((N, 16), F32),
                   jax.ShapeDtypeStruct((N, 32), F32)),
    )(ea_packed, A12k, a12k, B12k, c12k, x, ntw, ntb)


def _xt1_body(x_ref, w_ref, b_ref, out_ref):
    out_ref[...] = x_ref[...] @ w_ref[...] + b_ref[...]


def _tc_xt1(x, w, b):
    BN = 2000
    return pl.pallas_call(
        _xt1_body,
        grid=(N // BN,),
        in_specs=[
            pl.BlockSpec((BN, 128), lambda i: (i, 0)),
            pl.BlockSpec((128, 16), lambda i: (0, 0)),
            pl.BlockSpec((1, 16), lambda i: (0, 0)),
        ],
        out_specs=pl.BlockSpec((BN, 16), lambda i: (i, 0)),
        out_shape=jax.ShapeDtypeStruct((N, 16), F32),
    )(x, w, b)


def _combine1_body(outA_ref, xt1_ref, um1_ref, um1b_ref,
                   um2_ref, um2b_ref, nt2_ref, nt2b_ref, xt2_ref, sm2_ref):
    accA = outA_ref[0] + outA_ref[1]
    aggr1 = accA[:, :16]
    sm2_ref[...] = accA[:, 16:]
    xt1 = xt1_ref[...]
    h = jnp.maximum(xt1 @ um1_ref[:16, :] + aggr1 @ um1_ref[16:, :] + um1b_ref[...], 0.0)
    x1 = jnp.maximum(h @ um2_ref[...] + um2b_ref[...], 0.0)
    xt2_ref[...] = x1 @ nt2_ref[...] + nt2b_ref[...]


def _tc_combine1(outA, xt1, um1, um1b, um2, um2b, nt2, nt2b):
    return pl.pallas_call(
        _combine1_body,
        out_shape=(jax.ShapeDtypeStruct((N, 16), F32),
                   jax.ShapeDtypeStruct((N, 16), F32)),
    )(outA, xt1, um1, um1b, um2, um2b, nt2, nt2b)


def _combine2_body(outB2_ref, sm2_ref, xt2_ref, um1_ref, um1b_ref,
                   um2_ref, um2b_ref, fcw_ref, fcb_ref, out_ref):
    aggr2 = outB2_ref[0] + outB2_ref[1] + sm2_ref[...]
    xt2 = xt2_ref[...]
    h = jnp.maximum(xt2 @ um1_ref[:16, :] + aggr2 @ um1_ref[16:, :] + um1b_ref[...], 0.0)
    x2 = jnp.maximum(h @ um2_ref[...] + um2b_ref[...], 0.0)
    pooled = jnp.sum(x2, axis=0, keepdims=True) * (1.0 / N)
    out_ref[...] = pooled @ fcw_ref[...] + fcb_ref[...]


def _tc_combine2(outB2, sm2, xt2, um1, um1b, um2, um2b, fcw, fcb):
    return pl.pallas_call(
        _combine2_body,
        out_shape=jax.ShapeDtypeStruct((1, 1), F32),
    )(outB2, sm2, xt2, um1, um1b, um2, um2b, fcw, fcb)


def kernel(x, edge_index, edge_attr, batch, params):
    p1, p2 = params['conv1'], params['conv2']

    A1 = p1['et_w'] @ p1['em1_w']
    a1 = p1['et_b'] @ p1['em1_w'] + p1['em1_b']
    A2 = p2['et_w'] @ p2['em1_w']
    a2 = p2['et_b'] @ p2['em1_w'] + p2['em1_b']
    A12 = jnp.concatenate([A1, A2], axis=1)
    a12 = jnp.concatenate([a1, a2])[None, :]
    Z16 = jnp.zeros((16, 16), F32)
    B12 = jnp.block([[p1['em2_w'], Z16], [Z16, p2['em2_w']]])
    c12 = jnp.concatenate([p1['em2_b'], p2['em2_b']])[None, :]
    I8 = jnp.eye(8, dtype=F32)
    A12k = jnp.kron(I8, A12)
    B12k = jnp.kron(I8, B12)
    a12k = jnp.tile(a12, (1, 8))
    c12k = jnp.tile(c12, (1, 8))

    src, dst = edge_index[0], edge_index[1]
    src3d = src[:REMBASE].reshape(NGROUPS, 8, CHUNK)
    dst3d = dst[:REMBASE].reshape(NGROUPS, 8, CHUNK)
    srcr = src[REMBASE:].reshape(REM_CHUNKS, 1, CHUNK)
    dstr = dst[REMBASE:].reshape(REM_CHUNKS, 1, CHUNK)
    zA = jnp.zeros((N, 32), F32)
    zB = jnp.zeros((N, 16), F32)

    me12p, xt1, xtp = _tc_edge_mlp(edge_attr.reshape(E // 8, 128),
                                   A12k, a12k, B12k, c12k,
                                   x, p1['nt_w'], p1['nt_b'][None, :])
    me12 = me12p.reshape(E, 32)

    outA = _sc_pass1(src3d, dst3d, srcr, dstr, xtp, me12, zA)

    xt2, sm2 = _tc_combine1(outA, xt1,
                            p1['um1_w'], p1['um1_b'][None, :],
                            p1['um2_w'], p1['um2_b'][None, :],
                            p2['nt_w'], p2['nt_b'][None, :])

    outB2 = _sc_pass2(src3d, dst3d, srcr, dstr, xt2, zB)

    out = _tc_combine2(outB2, sm2, xt2,
                       p2['um1_w'], p2['um1_b'][None, :],
                       p2['um2_w'], p2['um2_b'][None, :],
                       params['fc_w'], params['fc_b'][None, :])
    return out
